# baseline (speedup 1.0000x reference)
import jax
import jax.numpy as jnp
from jax import lax
from jax.experimental import pallas as pl
from jax.experimental.pallas import tpu as pltpu
from jax.experimental.pallas import tpu_sc as plsc

N_NODES = 10000
NP = 10240
HALF = 128
NC = 2
NS = 16
LANES = 16
CHUNK = 64
CH_PER_TILE = 160
CH_HALF = CH_PER_TILE // 2
EP = NS * CH_PER_TILE * CHUNK
ROWS_PER_TILE = NP // NS
R_BLK = 10240


def _sc_mesh():
    return plsc.VectorSubcoreMesh(
        core_axis_name="c", subcore_axis_name="s",
        num_cores=NC, num_subcores=NS)


def _deg_body(dst_hbm, out_hbm, dstv, degv):
    c = lax.axis_index("c")
    s = lax.axis_index("s")
    wid = c * NS + s
    epw = EP // (NC * NS)
    pltpu.sync_copy(dst_hbm.at[pl.ds(wid * epw, epw)], dstv)

    zero = jnp.zeros((LANES,), jnp.float32)

    def zbody(k, carry):
        degv[pl.ds(k * LANES, LANES)] = zero
        return carry

    lax.fori_loop(0, NP // LANES, zbody, 0)

    one = jnp.ones((LANES,), jnp.float32)

    def abody(k, carry):
        idx = dstv[pl.ds(k * LANES, LANES)]
        plsc.addupdate_scatter(degv, [idx], one)
        return carry

    lax.fori_loop(0, epw // LANES, abody, 0)
    pltpu.sync_copy(degv, out_hbm.at[pl.ds(wid * NP, NP)])


def _deg_partials(dst_flat):
    k = pl.kernel(
        _deg_body,
        out_type=jax.ShapeDtypeStruct((NC * NS * NP,), jnp.float32),
        mesh=_sc_mesh(),
        compiler_params=pltpu.CompilerParams(needs_layout_passes=False),
        scratch_types=[
            pltpu.VMEM((EP // (NC * NS),), jnp.int32),
            pltpu.VMEM((NP,), jnp.float32),
        ])
    return k(dst_flat)


def _prop_body(u_hbm, src_hbm, dst_hbm, out_hbm, srcv, dstv,
               b0, b1, b2, b3, accsh,
               ga, gb, gc, gd, sa, sb, sc, sd):
    c = lax.axis_index("c")
    s = lax.axis_index("s")
    bufs = [b0, b1, b2, b3]
    gs = [ga, gb, gc, gd]
    ss = [sa, sb, sc, sd]
    base = c * NP + s * ROWS_PER_TILE
    pltpu.sync_copy(u_hbm.at[pl.ds(base, ROWS_PER_TILE)],
                    accsh.at[pl.ds(s * ROWS_PER_TILE, ROWS_PER_TILE)])
    plsc.subcore_barrier()

    def src_idx(row, col):
        return srcv.at[row, pl.ds(col * 64, 64)]

    for h in range(2):
        pltpu.sync_copy(
            src_hbm.at[pl.ds(c * (NS * CH_PER_TILE // 2)
                             + s * (CH_PER_TILE // 2) + h * (CH_HALF // 2),
                             CH_HALF // 2)], srcv)
        pltpu.sync_copy(
            dst_hbm.at[pl.ds(s * CH_PER_TILE + h * CH_HALF, CH_HALF)], dstv)

        pltpu.async_copy(u_hbm.at[src_idx(0, 0)], bufs[0], gs[0])
        pltpu.async_copy(u_hbm.at[src_idx(0, 1)], bufs[1], gs[1])

        def body(ii, carry):
            for k in range(4):
                j = 4 * ii + k
                kn = (k + 2) % 4
                pltpu.make_async_copy(
                    u_hbm.at[src_idx(2 * ii + k // 2, k % 2)],
                    bufs[k], gs[k]).wait()
                pltpu.async_copy(bufs[k], accsh.at[dstv.at[j]], ss[k],
                                 add=True)

                def retire(jj=j, kkn=kn):
                    pltpu.make_async_copy(
                        bufs[kkn], accsh.at[dstv.at[jj - 2]],
                        ss[kkn]).wait()

                def prefetch(ii=ii, k=k, kkn=kn):
                    pltpu.async_copy(
                        u_hbm.at[src_idx(2 * ii + (k + 2) // 2, k % 2)],
                        bufs[kkn], gs[kkn])

                if k < 2:
                    pl.when(ii >= 1)(retire)
                    prefetch()
                else:
                    retire()
                    pl.when(ii < CH_HALF // 4 - 1)(prefetch)
            return carry

        lax.fori_loop(0, CH_HALF // 4, body, 0)
        pltpu.make_async_copy(
            bufs[2], accsh.at[dstv.at[CH_HALF - 2]], ss[2]).wait()
        pltpu.make_async_copy(
            bufs[3], accsh.at[dstv.at[CH_HALF - 1]], ss[3]).wait()
    plsc.subcore_barrier()
    pltpu.sync_copy(accsh.at[pl.ds(s * ROWS_PER_TILE, ROWS_PER_TILE)],
                    out_hbm.at[pl.ds(base, ROWS_PER_TILE)])


def _propagate(u_flat, src_b, dst_b):
    k = pl.kernel(
        _prop_body,
        out_type=jax.ShapeDtypeStruct((NC * NP, HALF), jnp.float32),
        mesh=_sc_mesh(),
        scratch_types=(
            [pltpu.VMEM((CH_HALF // 2, 2 * CHUNK), jnp.int32),
             pltpu.VMEM((CH_HALF, CHUNK), jnp.int32)]
            + [pltpu.VMEM((CHUNK, HALF), jnp.float32)] * 4
            + [pltpu.VMEM_SHARED((NP, HALF), jnp.float32)]
            + [pltpu.SemaphoreType.DMA] * 8
        ))
    return k(u_flat, src_b, dst_b)


def _scale_split_body(x_ref, d8_ref, u_ref):
    u_ref[0] = x_ref[...] * d8_ref[:, 0:1]


def _scale_split(xp, dis8):
    return pl.pallas_call(
        _scale_split_body,
        grid=(NC, NP // R_BLK),
        in_specs=[
            pl.BlockSpec((R_BLK, HALF), lambda j, i: (i, j)),
            pl.BlockSpec((R_BLK, 8), lambda j, i: (i, 0)),
        ],
        out_specs=pl.BlockSpec((1, R_BLK, HALF), lambda j, i: (j, i, 0)),
        out_shape=jax.ShapeDtypeStruct((NC, NP, HALF), jnp.float32),
    )(xp, dis8)


def _mlp_body(sl_ref, sr_ref, d8_ref, W1_ref, b1_ref, W2_ref, u2_ref):
    dis = d8_ref[:, 0:1]
    p = jnp.concatenate([sl_ref[0], sr_ref[0]], axis=1) * dis
    h = jnp.dot(p.astype(jnp.bfloat16), W1_ref[...].astype(jnp.bfloat16),
                preferred_element_type=jnp.float32)
    h = jnp.maximum(h + b1_ref[...], 0.0)
    m = jnp.dot(h.astype(jnp.bfloat16), W2_ref[...].astype(jnp.bfloat16),
                preferred_element_type=jnp.float32)
    u2 = m * dis
    u2_ref[0] = u2[:, :HALF]
    u2_ref[1] = u2[:, HALF:]


def _mlp(s1, dis8, W1, b1, W2):
    return pl.pallas_call(
        _mlp_body,
        grid=(NP // R_BLK,),
        in_specs=[
            pl.BlockSpec((1, R_BLK, HALF), lambda i: (0, i, 0)),
            pl.BlockSpec((1, R_BLK, HALF), lambda i: (1, i, 0)),
            pl.BlockSpec((R_BLK, 8), lambda i: (i, 0)),
            pl.BlockSpec(W1.shape, lambda i: (0, 0)),
            pl.BlockSpec((1, b1.shape[1]), lambda i: (0, 0)),
            pl.BlockSpec(W2.shape, lambda i: (0, 0)),
        ],
        out_specs=pl.BlockSpec((NC, R_BLK, HALF), lambda i: (0, i, 0)),
        out_shape=jax.ShapeDtypeStruct((NC, NP, HALF), jnp.float32),
    )(s1, s1, dis8, W1, b1, W2)


def _finish_body(sl_ref, sr_ref, d8_ref, b2_ref, o_ref):
    dis = d8_ref[:, 0:1]
    o_ref[...] = (jnp.concatenate([sl_ref[0], sr_ref[0]], axis=1) * dis
                  + b2_ref[...])


def _finish(s2, dis8, b2):
    return pl.pallas_call(
        _finish_body,
        grid=(NP // R_BLK,),
        in_specs=[
            pl.BlockSpec((1, R_BLK, HALF), lambda i: (0, i, 0)),
            pl.BlockSpec((1, R_BLK, HALF), lambda i: (1, i, 0)),
            pl.BlockSpec((R_BLK, 8), lambda i: (i, 0)),
            pl.BlockSpec((1, b2.shape[1]), lambda i: (0, 0)),
        ],
        out_specs=pl.BlockSpec((R_BLK, 2 * HALF), lambda i: (i, 0)),
        out_shape=jax.ShapeDtypeStruct((N_NODES, 2 * HALF), jnp.float32),
    )(s2, s2, dis8, b2)


def kernel(x, edge_index, W1, b1, W2, b2):
    src = edge_index[0]
    dst = edge_index[1]
    e = src.shape[0]
    padidx = jnp.full((EP - e,), NP - 1, jnp.int32)
    srcp = jnp.concatenate([src, padidx])
    dstp = jnp.concatenate([dst, padidx])

    degp = _deg_partials(dstp).reshape(NC * NS, NP)
    deg = 1.0 + degp.sum(axis=0)
    dis8 = jnp.broadcast_to(lax.rsqrt(deg)[:, None], (NP, 8))

    src2d = srcp.reshape(NS * CH_PER_TILE // 2, 2 * CHUNK)
    src_b = jnp.concatenate([src2d, src2d + NP], axis=0)
    dst_b = dstp.reshape(NS * CH_PER_TILE, CHUNK)

    u1 = _scale_split(x, dis8).reshape(NC * NP, HALF)
    s1 = _propagate(u1, src_b, dst_b).reshape(NC, NP, HALF)
    u2 = _mlp(s1, dis8, W1, b1.reshape(1, -1), W2).reshape(NC * NP, HALF)
    s2 = _propagate(u2, src_b, dst_b).reshape(NC, NP, HALF)
    return _finish(s2, dis8, b2.reshape(1, -1))


# speedup vs baseline: 1.0036x; 1.0036x over previous
import jax
import jax.numpy as jnp
from jax import lax
from jax.experimental import pallas as pl
from jax.experimental.pallas import tpu as pltpu
from jax.experimental.pallas import tpu_sc as plsc

N_NODES = 10000
NP = 10240
HALF = 128
NC = 2
NS = 16
LANES = 16
CHUNK = 64
CH_PER_TILE = 160
CH_HALF = CH_PER_TILE // 2
EP = NS * CH_PER_TILE * CHUNK
ROWS_PER_TILE = NP // NS
R_BLK = 5120


def _sc_mesh():
    return plsc.VectorSubcoreMesh(
        core_axis_name="c", subcore_axis_name="s",
        num_cores=NC, num_subcores=NS)


def _deg_body(dst_hbm, out_hbm, dstv, degv):
    c = lax.axis_index("c")
    s = lax.axis_index("s")
    wid = c * NS + s
    epw = EP // (NC * NS)
    pltpu.sync_copy(dst_hbm.at[pl.ds(wid * epw, epw)], dstv)

    zero = jnp.zeros((LANES,), jnp.float32)

    def zbody(k, carry):
        degv[pl.ds(k * LANES, LANES)] = zero
        return carry

    lax.fori_loop(0, NP // LANES, zbody, 0)

    one = jnp.ones((LANES,), jnp.float32)

    def abody(k, carry):
        idx = dstv[pl.ds(k * LANES, LANES)]
        plsc.addupdate_scatter(degv, [idx], one)
        return carry

    lax.fori_loop(0, epw // LANES, abody, 0)
    pltpu.sync_copy(degv, out_hbm.at[pl.ds(wid * NP, NP)])


def _deg_partials(dst_flat):
    k = pl.kernel(
        _deg_body,
        out_type=jax.ShapeDtypeStruct((NC * NS * NP,), jnp.float32),
        mesh=_sc_mesh(),
        compiler_params=pltpu.CompilerParams(needs_layout_passes=False),
        scratch_types=[
            pltpu.VMEM((EP // (NC * NS),), jnp.int32),
            pltpu.VMEM((NP,), jnp.float32),
        ])
    return k(dst_flat)


def _prop_body(u_hbm, src_hbm, dst_hbm, out_hbm, srcv, dstv,
               b0, b1, b2, b3, accsh,
               ga, gb, gc, gd, sa, sb, sc, sd):
    c = lax.axis_index("c")
    s = lax.axis_index("s")
    bufs = [b0, b1, b2, b3]
    gs = [ga, gb, gc, gd]
    ss = [sa, sb, sc, sd]
    base = c * NP + s * ROWS_PER_TILE
    pltpu.sync_copy(u_hbm.at[pl.ds(base, ROWS_PER_TILE)],
                    accsh.at[pl.ds(s * ROWS_PER_TILE, ROWS_PER_TILE)])
    plsc.subcore_barrier()

    def src_idx(row, col):
        return srcv.at[row, pl.ds(col * 64, 64)]

    for h in range(2):
        pltpu.sync_copy(
            src_hbm.at[pl.ds(c * (NS * CH_PER_TILE // 2)
                             + s * (CH_PER_TILE // 2) + h * (CH_HALF // 2),
                             CH_HALF // 2)], srcv)
        pltpu.sync_copy(
            dst_hbm.at[pl.ds(s * CH_PER_TILE + h * CH_HALF, CH_HALF)], dstv)

        pltpu.async_copy(u_hbm.at[src_idx(0, 0)], bufs[0], gs[0])
        pltpu.async_copy(u_hbm.at[src_idx(0, 1)], bufs[1], gs[1])

        def body(ii, carry):
            for k in range(4):
                j = 4 * ii + k
                kn = (k + 2) % 4
                pltpu.make_async_copy(
                    u_hbm.at[src_idx(2 * ii + k // 2, k % 2)],
                    bufs[k], gs[k]).wait()
                pltpu.async_copy(bufs[k], accsh.at[dstv.at[j]], ss[k],
                                 add=True)

                def retire(jj=j, kkn=kn):
                    pltpu.make_async_copy(
                        bufs[kkn], accsh.at[dstv.at[jj - 2]],
                        ss[kkn]).wait()

                def prefetch(ii=ii, k=k, kkn=kn):
                    pltpu.async_copy(
                        u_hbm.at[src_idx(2 * ii + (k + 2) // 2, k % 2)],
                        bufs[kkn], gs[kkn])

                if k < 2:
                    pl.when(ii >= 1)(retire)
                    prefetch()
                else:
                    retire()
                    pl.when(ii < CH_HALF // 4 - 1)(prefetch)
            return carry

        lax.fori_loop(0, CH_HALF // 4, body, 0)
        pltpu.make_async_copy(
            bufs[2], accsh.at[dstv.at[CH_HALF - 2]], ss[2]).wait()
        pltpu.make_async_copy(
            bufs[3], accsh.at[dstv.at[CH_HALF - 1]], ss[3]).wait()
    plsc.subcore_barrier()
    pltpu.sync_copy(accsh.at[pl.ds(s * ROWS_PER_TILE, ROWS_PER_TILE)],
                    out_hbm.at[pl.ds(base, ROWS_PER_TILE)])


def _propagate(u_flat, src_b, dst_b):
    k = pl.kernel(
        _prop_body,
        out_type=jax.ShapeDtypeStruct((NC * NP, HALF), jnp.float32),
        mesh=_sc_mesh(),
        scratch_types=(
            [pltpu.VMEM((CH_HALF // 2, 2 * CHUNK), jnp.int32),
             pltpu.VMEM((CH_HALF, CHUNK), jnp.int32)]
            + [pltpu.VMEM((CHUNK, HALF), jnp.float32)] * 4
            + [pltpu.VMEM_SHARED((NP, HALF), jnp.float32)]
            + [pltpu.SemaphoreType.DMA] * 8
        ))
    return k(u_flat, src_b, dst_b)


def _scale_split_body(x_ref, d8_ref, u_ref):
    u_ref[0] = x_ref[...] * d8_ref[:, 0:1]


def _scale_split(xp, dis8):
    return pl.pallas_call(
        _scale_split_body,
        grid=(NC, NP // R_BLK),
        in_specs=[
            pl.BlockSpec((R_BLK, HALF), lambda j, i: (i, j)),
            pl.BlockSpec((R_BLK, 8), lambda j, i: (i, 0)),
        ],
        out_specs=pl.BlockSpec((1, R_BLK, HALF), lambda j, i: (j, i, 0)),
        out_shape=jax.ShapeDtypeStruct((NC, NP, HALF), jnp.float32),
    )(xp, dis8)


def _mlp_body(sl_ref, sr_ref, d8_ref, W1_ref, b1_ref, W2_ref, u2_ref):
    dis = d8_ref[:, 0:1]
    p = jnp.concatenate([sl_ref[0], sr_ref[0]], axis=1) * dis
    h = jnp.dot(p.astype(jnp.bfloat16), W1_ref[...].astype(jnp.bfloat16),
                preferred_element_type=jnp.float32)
    h = jnp.maximum(h + b1_ref[...], 0.0)
    m = jnp.dot(h.astype(jnp.bfloat16), W2_ref[...].astype(jnp.bfloat16),
                preferred_element_type=jnp.float32)
    u2 = m * dis
    u2_ref[0] = u2[:, :HALF]
    u2_ref[1] = u2[:, HALF:]


def _mlp(s1, dis8, W1, b1, W2):
    return pl.pallas_call(
        _mlp_body,
        grid=(NP // R_BLK,),
        in_specs=[
            pl.BlockSpec((1, R_BLK, HALF), lambda i: (0, i, 0)),
            pl.BlockSpec((1, R_BLK, HALF), lambda i: (1, i, 0)),
            pl.BlockSpec((R_BLK, 8), lambda i: (i, 0)),
            pl.BlockSpec(W1.shape, lambda i: (0, 0)),
            pl.BlockSpec((1, b1.shape[1]), lambda i: (0, 0)),
            pl.BlockSpec(W2.shape, lambda i: (0, 0)),
        ],
        out_specs=pl.BlockSpec((NC, R_BLK, HALF), lambda i: (0, i, 0)),
        out_shape=jax.ShapeDtypeStruct((NC, NP, HALF), jnp.float32),
    )(s1, s1, dis8, W1, b1, W2)


def _finish_body(sl_ref, sr_ref, d8_ref, b2_ref, o_ref):
    dis = d8_ref[:, 0:1]
    o_ref[...] = (jnp.concatenate([sl_ref[0], sr_ref[0]], axis=1) * dis
                  + b2_ref[...])


def _finish(s2, dis8, b2):
    return pl.pallas_call(
        _finish_body,
        grid=(NP // R_BLK,),
        in_specs=[
            pl.BlockSpec((1, R_BLK, HALF), lambda i: (0, i, 0)),
            pl.BlockSpec((1, R_BLK, HALF), lambda i: (1, i, 0)),
            pl.BlockSpec((R_BLK, 8), lambda i: (i, 0)),
            pl.BlockSpec((1, b2.shape[1]), lambda i: (0, 0)),
        ],
        out_specs=pl.BlockSpec((R_BLK, 2 * HALF), lambda i: (i, 0)),
        out_shape=jax.ShapeDtypeStruct((N_NODES, 2 * HALF), jnp.float32),
    )(s2, s2, dis8, b2)


def kernel(x, edge_index, W1, b1, W2, b2):
    src = edge_index[0]
    dst = edge_index[1]
    e = src.shape[0]
    padidx = jnp.full((EP - e,), NP - 1, jnp.int32)
    srcp = jnp.concatenate([src, padidx])
    dstp = jnp.concatenate([dst, padidx])

    degp = _deg_partials(dstp).reshape(NC * NS, NP)
    deg = 1.0 + degp.sum(axis=0)
    dis8 = jnp.broadcast_to(lax.rsqrt(deg)[:, None], (NP, 8))

    src2d = srcp.reshape(NS * CH_PER_TILE // 2, 2 * CHUNK)
    src_b = jnp.concatenate([src2d, src2d + NP], axis=0)
    dst_b = dstp.reshape(NS * CH_PER_TILE, CHUNK)

    u1 = _scale_split(x, dis8).reshape(NC * NP, HALF)
    s1 = _propagate(u1, src_b, dst_b).reshape(NC, NP, HALF)
    u2 = _mlp(s1, dis8, W1, b1.reshape(1, -1), W2).reshape(NC * NP, HALF)
    s2 = _propagate(u2, src_b, dst_b).reshape(NC, NP, HALF)
    return _finish(s2, dis8, b2.reshape(1, -1))
